# baseline (speedup 1.0000x reference)
import jax
import jax.numpy as jnp
from jax import lax
from jax.experimental import pallas as pl
from jax.experimental.pallas import tpu as pltpu
from jax.experimental.pallas import tpu_sc as plsc

N_NODES = 10000
N_EDGES = 320000
F = 128

NP = 10240
NC, NS = 2, 16
NW = NC * NS
CH = 128
NPASS = 4
CPP = 20
CPW = NPASS * CPP
EPW = CPW * CH
EP = NW * EPW
RPT = NP // NS

_sc_mesh = plsc.VectorSubcoreMesh(
    core_axis_name="c", subcore_axis_name="s", num_cores=NC, num_subcores=NS)


def _dense1_body(x_ref, wl_ref, wr_ref, b1_ref, y_ref, z_ref):
    x = x_ref[...]
    dn = (((1,), (1,)), ((), ()))
    y_ref[...] = lax.dot_general(x, wl_ref[...], dn,
                                 preferred_element_type=jnp.float32)
    z_ref[...] = lax.dot_general(x, wr_ref[...], dn,
                                 preferred_element_type=jnp.float32) + b1_ref[...]


_dense1 = pl.pallas_call(
    _dense1_body,
    out_shape=(jax.ShapeDtypeStruct((NP, F), jnp.float32),
               jax.ShapeDtypeStruct((NP, F), jnp.float32)),
)


def _sc_agg_body(y_hbm, srcs_hbm, dsts_hbm, zeros_hbm, agg_hbm, degp_hbm,
                 src_v, dst_v, rows_v, hist_v, acc_s,
                 gsem0, gsem1, ssem0, ssem1):
    cid = lax.axis_index("c")
    sid = lax.axis_index("s")
    wid = sid * NC + cid
    row0 = pl.multiple_of(sid * RPT, RPT)

    pltpu.sync_copy(zeros_hbm.at[pl.ds(row0, RPT)], acc_s.at[pl.ds(row0, RPT)])

    def _zbody(i, carry):
        hist_v[pl.ds(pl.multiple_of(i * 16, 16), 16)] = jnp.zeros((16,), jnp.float32)
        return carry
    lax.fori_loop(0, NP // 16, _zbody, 0)

    ones16 = jnp.ones((16,), jnp.float32)

    plsc.subcore_barrier()

    gsem = (gsem0, gsem1)
    ssem = (ssem0, ssem1)

    def _gather(c, b):
        pltpu.async_copy(y_hbm.at[src_v.at[c]], rows_v.at[b], gsem[b])

    def _gwait(c, b):
        pltpu.make_async_copy(y_hbm.at[src_v.at[c]], rows_v.at[b],
                              gsem[b]).wait()

    def _scatter(c, b):
        pltpu.async_copy(rows_v.at[b], acc_s.at[dst_v.at[c]], ssem[b],
                         add=True)

    def _swait(c, b):
        pltpu.make_async_copy(rows_v.at[b], acc_s.at[dst_v.at[c]],
                              ssem[b]).wait()

    def _hist(c):
        for j in range(CH // 16):
            dstv = dst_v[c, pl.ds(j * 16, 16)]
            plsc.addupdate_scatter(hist_v, [dstv], ones16)

    for p in range(NPASS):
        pltpu.sync_copy(srcs_hbm.at[wid, p], src_v)
        pltpu.sync_copy(dsts_hbm.at[wid, p], dst_v)

        _gather(0, 0)
        _hist(0)
        _gwait(0, 0)
        _scatter(0, 0)
        _gather(1, 1)

        def _ebody(i, carry):
            for off, b in ((1, 1), (2, 0)):
                c = 2 * i + off
                _hist(c)
                _gwait(c, b)
                _scatter(c, b)
                _swait(c - 1, 1 - b)
                _gather(c + 1, 1 - b)
            return carry
        lax.fori_loop(0, (CPP - 2) // 2, _ebody, 0)

        _hist(CPP - 1)
        _gwait(CPP - 1, 1)
        _scatter(CPP - 1, 1)
        _swait(CPP - 2, 0)
        _swait(CPP - 1, 1)

    plsc.subcore_barrier()

    pltpu.sync_copy(acc_s.at[pl.ds(row0, RPT)], agg_hbm.at[cid, pl.ds(row0, RPT)])
    pltpu.sync_copy(hist_v, degp_hbm.at[wid])


_sc_agg = pl.kernel(
    _sc_agg_body,
    out_type=(jax.ShapeDtypeStruct((NC, NP, F), jnp.float32),
              jax.ShapeDtypeStruct((NW, NP), jnp.float32)),
    mesh=_sc_mesh,
    scratch_types=[
        pltpu.VMEM((CPP, CH), jnp.int32),
        pltpu.VMEM((CPP, CH), jnp.int32),
        pltpu.VMEM((2, CH, F), jnp.float32),
        pltpu.VMEM((NP,), jnp.float32),
        pltpu.VMEM_SHARED((NP, F), jnp.float32),
        pltpu.SemaphoreType.DMA,
        pltpu.SemaphoreType.DMA,
        pltpu.SemaphoreType.DMA,
        pltpu.SemaphoreType.DMA,
    ],
    compiler_params=pltpu.CompilerParams(needs_layout_passes=False),
)


def _mid_body(aggp_ref, degt_ref, z_ref, w2l_ref, w2r_ref, b2_ref,
              s_ref, t_ref, inv_ref):
    deg = jnp.sum(degt_ref[...], axis=1, keepdims=True)
    inv = 1.0 / jnp.maximum(deg, 1.0)
    agg = aggp_ref[0] + aggp_ref[1]
    h = jnp.maximum(agg * inv + z_ref[...], 0.0)
    rows = lax.broadcasted_iota(jnp.int32, (NP, 1), 0)
    live = rows < N_NODES
    s = jnp.sum(h * w2l_ref[...], axis=1, keepdims=True)
    t = jnp.sum(h * w2r_ref[...], axis=1, keepdims=True) + b2_ref[...]
    s_ref[...] = jnp.where(live, s, 0.0)
    t_ref[...] = t
    inv_ref[...] = inv


_mid = pl.pallas_call(
    _mid_body,
    out_shape=(jax.ShapeDtypeStruct((NP, 1), jnp.float32),
               jax.ShapeDtypeStruct((NP, 1), jnp.float32),
               jax.ShapeDtypeStruct((NP, 1), jnp.float32)),
)


def _sc_scalar_body(s_hbm, srcs_hbm, dsts_hbm, parts_hbm,
                    s_v, acc_v, src_v, dst_v):
    cid = lax.axis_index("c")
    sid = lax.axis_index("s")
    wid = sid * NC + cid

    pltpu.sync_copy(s_hbm, s_v)
    pltpu.sync_copy(srcs_hbm.at[wid], src_v)
    pltpu.sync_copy(dsts_hbm.at[wid], dst_v)

    def _zbody(i, carry):
        acc_v[pl.ds(pl.multiple_of(i * 16, 16), 16)] = jnp.zeros((16,), jnp.float32)
        return carry
    lax.fori_loop(0, NP // 16, _zbody, 0)

    def _ebody(r, carry):
        for j in range(CH // 16):
            srcv = src_v[r, pl.ds(j * 16, 16)]
            dstv = dst_v[r, pl.ds(j * 16, 16)]
            vals = plsc.load_gather(s_v, [srcv])
            plsc.addupdate_scatter(acc_v, [dstv], vals)
        return carry
    lax.fori_loop(0, CPW, _ebody, 0)

    pltpu.sync_copy(acc_v, parts_hbm.at[wid])


_sc_scalar = pl.kernel(
    _sc_scalar_body,
    out_type=jax.ShapeDtypeStruct((NW, NP), jnp.float32),
    mesh=_sc_mesh,
    scratch_types=[
        pltpu.VMEM((NP,), jnp.float32),
        pltpu.VMEM((NP,), jnp.float32),
        pltpu.VMEM((CPW, CH), jnp.int32),
        pltpu.VMEM((CPW, CH), jnp.int32),
    ],
    compiler_params=pltpu.CompilerParams(needs_layout_passes=False),
)


GR = NP // 128


def _final_body(parts_ref, inv_ref, t_ref, out_ref):
    acc = parts_ref[0]
    for w in range(1, NW):
        acc = acc + parts_ref[w]
    out_ref[...] = acc * inv_ref[...] + t_ref[...]


_final = pl.pallas_call(
    _final_body,
    out_shape=jax.ShapeDtypeStruct((GR, 128), jnp.float32),
)


def kernel(x, edge_index, W1l, b1, W1r, W2l, b2, W2r):
    src = edge_index[0].astype(jnp.int32)
    dst = edge_index[1].astype(jnp.int32)
    pad = EP - N_EDGES
    ppw = pad // NW
    dums = N_NODES + (jnp.arange(ppw, dtype=jnp.int32) % (NP - N_NODES))
    dums = jnp.broadcast_to(dums, (NW, ppw))
    srcp = jnp.concatenate([src.reshape(NW, EPW - ppw), dums], axis=1
                           ).reshape(NW, NPASS, CPP, CH)
    dstp = jnp.concatenate([dst.reshape(NW, EPW - ppw), dums], axis=1
                           ).reshape(NW, NPASS, CPP, CH)
    xp = jnp.pad(x, ((0, NP - N_NODES), (0, 0)))
    zeros_np = jnp.zeros((NP, F), jnp.float32)

    y, z = _dense1(xp, W1l, W1r, b1)
    aggp, degp = _sc_agg(y, srcp, dstp, zeros_np)
    s2, t2, inv2 = _mid(aggp, degp.T, z, W2l, W2r, b2)
    parts = _sc_scalar(s2.reshape(NP), srcp.reshape(NW, CPW, CH),
                       dstp.reshape(NW, CPW, CH))
    res = _final(parts.reshape(NW, GR, 128),
                 inv2.reshape(GR, 128), t2.reshape(GR, 128))
    return res.reshape(NP, 1)[:N_NODES]


# speedup vs baseline: 1.0442x; 1.0442x over previous
import jax
import jax.numpy as jnp
from jax import lax
from jax.experimental import pallas as pl
from jax.experimental.pallas import tpu as pltpu
from jax.experimental.pallas import tpu_sc as plsc

N_NODES = 10000
N_EDGES = 320000
F = 128

NP = 10240
NC, NS = 2, 16
NW = NC * NS
CH = 128
NPASS = 2
CPP = 40
CPW = NPASS * CPP
EPW = CPW * CH
EP = NW * EPW
RPT = NP // NS

_sc_mesh = plsc.VectorSubcoreMesh(
    core_axis_name="c", subcore_axis_name="s", num_cores=NC, num_subcores=NS)


def _sc_agg_body(y_hbm, srcs_hbm, dsts_hbm, zeros_hbm, agg_hbm, deg_hbm,
                 src_v, dst_v, rows_v, ones_v, zd_v, acc_s, deg_s,
                 gsem0, gsem1, ssem0, ssem1, osem):
    cid = lax.axis_index("c")
    sid = lax.axis_index("s")
    wid = sid * NC + cid
    row0 = pl.multiple_of(sid * RPT, RPT)

    pltpu.sync_copy(zeros_hbm.at[pl.ds(row0, RPT)], acc_s.at[pl.ds(row0, RPT)])

    def _zbody(i, carry):
        zd_v[pl.ds(pl.multiple_of(i * 16, 16), 16)] = jnp.zeros((16,), jnp.float32)
        return carry
    lax.fori_loop(0, RPT // 16, _zbody, 0)
    pltpu.sync_copy(zd_v, deg_s.at[pl.ds(row0, RPT)])

    for j in range(CH // 16):
        ones_v[pl.ds(j * 16, 16)] = jnp.ones((16,), jnp.float32)

    plsc.subcore_barrier()

    gsem = (gsem0, gsem1)
    ssem = (ssem0, ssem1)

    def _gather(c, b):
        pltpu.async_copy(y_hbm.at[src_v.at[c]], rows_v.at[b], gsem[b])

    def _gwait(c, b):
        pltpu.make_async_copy(y_hbm.at[src_v.at[c]], rows_v.at[b],
                              gsem[b]).wait()

    def _scatter(c, b):
        pltpu.async_copy(rows_v.at[b], acc_s.at[dst_v.at[c]], ssem[b],
                         add=True)

    def _swait(c, b):
        pltpu.make_async_copy(rows_v.at[b], acc_s.at[dst_v.at[c]],
                              ssem[b]).wait()

    def _ones(c):
        pltpu.async_copy(ones_v, deg_s.at[dst_v.at[c]], osem, add=True)

    for p in range(NPASS):
        pltpu.sync_copy(srcs_hbm.at[wid, p], src_v)
        pltpu.sync_copy(dsts_hbm.at[wid, p], dst_v)

        _gather(0, 0)
        _gwait(0, 0)
        _scatter(0, 0)
        _ones(0)
        _gather(1, 1)

        def _ebody(i, carry):
            for off, b in ((1, 1), (2, 0)):
                c = 2 * i + off
                _gwait(c, b)
                _scatter(c, b)
                _ones(c)
                _swait(c - 1, 1 - b)
                _gather(c + 1, 1 - b)
            return carry
        lax.fori_loop(0, (CPP - 2) // 2, _ebody, 0)

        _gwait(CPP - 1, 1)
        _scatter(CPP - 1, 1)
        _ones(CPP - 1)
        _swait(CPP - 2, 0)
        _swait(CPP - 1, 1)
        for _ in range(CPP):
            pltpu.make_async_copy(ones_v, deg_s.at[dst_v.at[0]], osem).wait()

    plsc.subcore_barrier()

    pltpu.sync_copy(acc_s.at[pl.ds(row0, RPT)], agg_hbm.at[cid, pl.ds(row0, RPT)])
    pltpu.sync_copy(deg_s.at[pl.ds(row0, RPT)], deg_hbm.at[cid, pl.ds(row0, RPT)])


_sc_agg = pl.kernel(
    _sc_agg_body,
    out_type=(jax.ShapeDtypeStruct((NC, NP, F), jnp.float32),
              jax.ShapeDtypeStruct((NC, NP), jnp.float32)),
    mesh=_sc_mesh,
    scratch_types=[
        pltpu.VMEM((CPP, CH), jnp.int32),
        pltpu.VMEM((CPP, CH), jnp.int32),
        pltpu.VMEM((2, CH, F), jnp.float32),
        pltpu.VMEM((CH,), jnp.float32),
        pltpu.VMEM((RPT,), jnp.float32),
        pltpu.VMEM_SHARED((NP, F), jnp.float32),
        pltpu.VMEM_SHARED((NP,), jnp.float32),
        pltpu.SemaphoreType.DMA,
        pltpu.SemaphoreType.DMA,
        pltpu.SemaphoreType.DMA,
        pltpu.SemaphoreType.DMA,
        pltpu.SemaphoreType.DMA,
    ],
    compiler_params=pltpu.CompilerParams(needs_layout_passes=False),
)


def _mid_body(aggp_ref, degp_ref, x_ref, w1l_ref, b1_ref, w1r_ref,
              w2l_ref, w2r_ref, b2_ref, s_ref, t_ref, inv_ref):
    deg = degp_ref[0] + degp_ref[1]
    inv = 1.0 / jnp.maximum(deg, 1.0)
    agg = (aggp_ref[0] + aggp_ref[1]) * inv
    dn = (((1,), (1,)), ((), ()))
    z = lax.dot_general(agg, w1l_ref[...], dn,
                        preferred_element_type=jnp.float32)
    z = z + lax.dot_general(x_ref[...], w1r_ref[...], dn,
                            preferred_element_type=jnp.float32)
    h = jnp.maximum(z + b1_ref[...], 0.0)
    rows = lax.broadcasted_iota(jnp.int32, (NP, 1), 0)
    live = rows < N_NODES
    s = jnp.sum(h * w2l_ref[...], axis=1, keepdims=True)
    t = jnp.sum(h * w2r_ref[...], axis=1, keepdims=True) + b2_ref[...]
    s_ref[...] = jnp.where(live, s, 0.0)
    t_ref[...] = t
    inv_ref[...] = inv


_mid = pl.pallas_call(
    _mid_body,
    out_shape=(jax.ShapeDtypeStruct((NP, 1), jnp.float32),
               jax.ShapeDtypeStruct((NP, 1), jnp.float32),
               jax.ShapeDtypeStruct((NP, 1), jnp.float32)),
)


def _sc_scalar_body(s_hbm, srcs_hbm, dsts_hbm, parts_hbm,
                    s_v, acc_v, src_v, dst_v):
    cid = lax.axis_index("c")
    sid = lax.axis_index("s")
    wid = sid * NC + cid

    pltpu.sync_copy(s_hbm, s_v)
    for p in range(NPASS):
        pltpu.sync_copy(srcs_hbm.at[wid, p], src_v.at[p])
        pltpu.sync_copy(dsts_hbm.at[wid, p], dst_v.at[p])

    def _zbody(i, carry):
        acc_v[pl.ds(pl.multiple_of(i * 16, 16), 16)] = jnp.zeros((16,), jnp.float32)
        return carry
    lax.fori_loop(0, NP // 16, _zbody, 0)

    def _ebody(r, carry):
        for pp in range(NPASS):
            for j in range(CH // 16):
                srcv = src_v[pp, r, pl.ds(j * 16, 16)]
                dstv = dst_v[pp, r, pl.ds(j * 16, 16)]
                vals = plsc.load_gather(s_v, [srcv])
                plsc.addupdate_scatter(acc_v, [dstv], vals)
        return carry
    lax.fori_loop(0, CPP, _ebody, 0)

    pltpu.sync_copy(acc_v, parts_hbm.at[wid])


_sc_scalar = pl.kernel(
    _sc_scalar_body,
    out_type=jax.ShapeDtypeStruct((NW, NP), jnp.float32),
    mesh=_sc_mesh,
    scratch_types=[
        pltpu.VMEM((NP,), jnp.float32),
        pltpu.VMEM((NP,), jnp.float32),
        pltpu.VMEM((NPASS, CPP, CH), jnp.int32),
        pltpu.VMEM((NPASS, CPP, CH), jnp.int32),
    ],
    compiler_params=pltpu.CompilerParams(needs_layout_passes=False),
)


GR = NP // 128


def _final_body(parts_ref, inv_ref, t_ref, out_ref):
    acc = parts_ref[0]
    for w in range(1, NW):
        acc = acc + parts_ref[w]
    out_ref[...] = acc * inv_ref[...] + t_ref[...]


_final = pl.pallas_call(
    _final_body,
    out_shape=jax.ShapeDtypeStruct((GR, 128), jnp.float32),
)


def kernel(x, edge_index, W1l, b1, W1r, W2l, b2, W2r):
    src = edge_index[0].astype(jnp.int32)
    dst = edge_index[1].astype(jnp.int32)
    pad = EP - N_EDGES
    ppw = pad // NW
    dums = N_NODES + (jnp.arange(ppw, dtype=jnp.int32) % (NP - N_NODES))
    dums = jnp.broadcast_to(dums, (NW, ppw))
    srcp = jnp.concatenate([src.reshape(NW, EPW - ppw), dums], axis=1
                           ).reshape(NW, NPASS, CPP, CH)
    dstp = jnp.concatenate([dst.reshape(NW, EPW - ppw), dums], axis=1
                           ).reshape(NW, NPASS, CPP, CH)
    xp = jnp.pad(x, ((0, NP - N_NODES), (0, 0)))
    zeros_np = jnp.zeros((NP, F), jnp.float32)

    aggp, degp = _sc_agg(xp, srcp, dstp, zeros_np)
    s2, t2, inv2 = _mid(aggp, degp.reshape(NC, NP, 1), xp,
                        W1l, b1, W1r, W2l, W2r, b2)
    parts = _sc_scalar(s2.reshape(NP), srcp, dstp)
    res = _final(parts.reshape(NW, GR, 128),
                 inv2.reshape(GR, 128), t2.reshape(GR, 128))
    return res.reshape(NP, 1)[:N_NODES]


# speedup vs baseline: 1.0496x; 1.0052x over previous
import jax
import jax.numpy as jnp
from jax import lax
from jax.experimental import pallas as pl
from jax.experimental.pallas import tpu as pltpu
from jax.experimental.pallas import tpu_sc as plsc

N_NODES = 10000
N_EDGES = 320000
F = 128

NP = 10240
NC, NS = 2, 16
NW = NC * NS
CH = 128
NPASS = 2
CPP = 40
CPW = NPASS * CPP
EPW = CPW * CH
EP = NW * EPW
RPT = NP // NS

_sc_mesh = plsc.VectorSubcoreMesh(
    core_axis_name="c", subcore_axis_name="s", num_cores=NC, num_subcores=NS)


def _sc_agg_body(y_hbm, srcs_hbm, dsts_hbm, zeros_hbm, agg_hbm, deg_hbm,
                 src_v, dst_v, rows_v, ones_v, zd_v, acc_s, deg_s,
                 gsem0, gsem1, ssem0, ssem1, osem):
    cid = lax.axis_index("c")
    sid = lax.axis_index("s")
    wid = sid * NC + cid
    row0 = pl.multiple_of(sid * RPT, RPT)

    pltpu.sync_copy(zeros_hbm.at[pl.ds(row0, RPT)], acc_s.at[pl.ds(row0, RPT)])

    def _zbody(i, carry):
        zd_v[pl.ds(pl.multiple_of(i * 16, 16), 16)] = jnp.zeros((16,), jnp.float32)
        return carry
    lax.fori_loop(0, RPT // 16, _zbody, 0)
    pltpu.sync_copy(zd_v, deg_s.at[pl.ds(row0, RPT)])

    for j in range(CH // 16):
        ones_v[pl.ds(j * 16, 16)] = jnp.ones((16,), jnp.float32)

    plsc.subcore_barrier()

    gsem = (gsem0, gsem1)
    ssem = (ssem0, ssem1)

    def _gather(c, b):
        pltpu.async_copy(y_hbm.at[src_v.at[c]], rows_v.at[b], gsem[b])

    def _gwait(c, b):
        pltpu.make_async_copy(y_hbm.at[src_v.at[c]], rows_v.at[b],
                              gsem[b]).wait()

    def _scatter(c, b):
        pltpu.async_copy(rows_v.at[b], acc_s.at[dst_v.at[c]], ssem[b],
                         add=True)

    def _swait(c, b):
        pltpu.make_async_copy(rows_v.at[b], acc_s.at[dst_v.at[c]],
                              ssem[b]).wait()

    def _ones(c):
        pltpu.async_copy(ones_v, deg_s.at[dst_v.at[c]], osem, add=True)

    for p in range(NPASS):
        pltpu.sync_copy(srcs_hbm.at[wid, p], src_v)
        pltpu.sync_copy(dsts_hbm.at[wid, p], dst_v)

        _gather(0, 0)
        _gwait(0, 0)
        _scatter(0, 0)
        _ones(0)
        _gather(1, 1)

        def _ebody(i, carry):
            for off, b in ((1, 1), (2, 0)):
                c = 2 * i + off
                _gwait(c, b)
                _scatter(c, b)
                _ones(c)
                _swait(c - 1, 1 - b)
                _gather(c + 1, 1 - b)
            return carry
        lax.fori_loop(0, (CPP - 2) // 2, _ebody, 0)

        _gwait(CPP - 1, 1)
        _scatter(CPP - 1, 1)
        _ones(CPP - 1)
        _swait(CPP - 2, 0)
        _swait(CPP - 1, 1)
        for _ in range(CPP):
            pltpu.make_async_copy(ones_v, deg_s.at[dst_v.at[0]], osem).wait()

    plsc.subcore_barrier()

    pltpu.sync_copy(acc_s.at[pl.ds(row0, RPT)], agg_hbm.at[cid, pl.ds(row0, RPT)])
    pltpu.sync_copy(deg_s.at[pl.ds(row0, RPT)], deg_hbm.at[cid, pl.ds(row0, RPT)])


_sc_agg = pl.kernel(
    _sc_agg_body,
    out_type=(jax.ShapeDtypeStruct((NC, NP, F), jnp.float32),
              jax.ShapeDtypeStruct((NC, NP), jnp.float32)),
    mesh=_sc_mesh,
    scratch_types=[
        pltpu.VMEM((CPP, CH), jnp.int32),
        pltpu.VMEM((CPP, CH), jnp.int32),
        pltpu.VMEM((2, CH, F), jnp.float32),
        pltpu.VMEM((CH,), jnp.float32),
        pltpu.VMEM((RPT,), jnp.float32),
        pltpu.VMEM_SHARED((NP, F), jnp.float32),
        pltpu.VMEM_SHARED((NP,), jnp.float32),
        pltpu.SemaphoreType.DMA,
        pltpu.SemaphoreType.DMA,
        pltpu.SemaphoreType.DMA,
        pltpu.SemaphoreType.DMA,
        pltpu.SemaphoreType.DMA,
    ],
    compiler_params=pltpu.CompilerParams(needs_layout_passes=False),
)


def _mid_body(aggp_ref, degp_ref, x_ref, w1l_ref, b1_ref, w1r_ref,
              w2l_ref, s_ref, h_ref):
    deg = degp_ref[0] + degp_ref[1]
    inv = 1.0 / jnp.maximum(deg, 1.0)
    agg = (aggp_ref[0] + aggp_ref[1]) * inv
    dn = (((1,), (1,)), ((), ()))
    z = lax.dot_general(agg, w1l_ref[...], dn,
                        preferred_element_type=jnp.float32)
    z = z + lax.dot_general(x_ref[...], w1r_ref[...], dn,
                            preferred_element_type=jnp.float32)
    h = jnp.maximum(z + b1_ref[...], 0.0)
    rows = lax.broadcasted_iota(jnp.int32, (NP, 1), 0)
    live = rows < N_NODES
    s = jnp.sum(h * w2l_ref[...], axis=1, keepdims=True)
    s_ref[...] = jnp.where(live, s, 0.0)
    h_ref[...] = h


_mid = pl.pallas_call(
    _mid_body,
    out_shape=(jax.ShapeDtypeStruct((NP, 1), jnp.float32),
               jax.ShapeDtypeStruct((NP, F), jnp.float32)),
)


def _mid2_body(h_ref, w2r_ref, b2_ref, t_ref):
    t_ref[...] = (jnp.sum(h_ref[...] * w2r_ref[...], axis=1, keepdims=True)
                  + b2_ref[...])


_mid2 = pl.pallas_call(
    _mid2_body,
    out_shape=jax.ShapeDtypeStruct((NP, 1), jnp.float32),
)


def _sc_scalar_body(s_hbm, srcs_hbm, dsts_hbm, parts_hbm,
                    s_v, acc_v, src_v, dst_v):
    cid = lax.axis_index("c")
    sid = lax.axis_index("s")
    wid = sid * NC + cid

    pltpu.sync_copy(s_hbm, s_v)
    for p in range(NPASS):
        pltpu.sync_copy(srcs_hbm.at[wid, p], src_v.at[p])
        pltpu.sync_copy(dsts_hbm.at[wid, p], dst_v.at[p])

    def _zbody(i, carry):
        acc_v[pl.ds(pl.multiple_of(i * 16, 16), 16)] = jnp.zeros((16,), jnp.float32)
        return carry
    lax.fori_loop(0, NP // 16, _zbody, 0)

    def _ebody(r, carry):
        for pp in range(NPASS):
            for j in range(CH // 16):
                srcv = src_v[pp, r, pl.ds(j * 16, 16)]
                dstv = dst_v[pp, r, pl.ds(j * 16, 16)]
                vals = plsc.load_gather(s_v, [srcv])
                plsc.addupdate_scatter(acc_v, [dstv], vals)
        return carry
    lax.fori_loop(0, CPP, _ebody, 0)

    pltpu.sync_copy(acc_v, parts_hbm.at[wid])


_sc_scalar = pl.kernel(
    _sc_scalar_body,
    out_type=jax.ShapeDtypeStruct((NW, NP), jnp.float32),
    mesh=_sc_mesh,
    scratch_types=[
        pltpu.VMEM((NP,), jnp.float32),
        pltpu.VMEM((NP,), jnp.float32),
        pltpu.VMEM((NPASS, CPP, CH), jnp.int32),
        pltpu.VMEM((NPASS, CPP, CH), jnp.int32),
    ],
    compiler_params=pltpu.CompilerParams(needs_layout_passes=False),
)


GR = NP // 128


def _final_body(parts_ref, degp_ref, t_ref, out_ref):
    acc = parts_ref[0]
    for w in range(1, NW):
        acc = acc + parts_ref[w]
    deg = degp_ref[0] + degp_ref[1]
    inv = 1.0 / jnp.maximum(deg, 1.0)
    out_ref[...] = acc * inv + t_ref[...]


_final = pl.pallas_call(
    _final_body,
    out_shape=jax.ShapeDtypeStruct((GR, 128), jnp.float32),
)


def kernel(x, edge_index, W1l, b1, W1r, W2l, b2, W2r):
    src = edge_index[0].astype(jnp.int32)
    dst = edge_index[1].astype(jnp.int32)
    pad = EP - N_EDGES
    ppw = pad // NW
    dums = N_NODES + (jnp.arange(ppw, dtype=jnp.int32) % (NP - N_NODES))
    dums = jnp.broadcast_to(dums, (NW, ppw))
    srcp = jnp.concatenate([src.reshape(NW, EPW - ppw), dums], axis=1
                           ).reshape(NW, NPASS, CPP, CH)
    dstp = jnp.concatenate([dst.reshape(NW, EPW - ppw), dums], axis=1
                           ).reshape(NW, NPASS, CPP, CH)
    xp = jnp.pad(x, ((0, NP - N_NODES), (0, 0)))
    zeros_np = jnp.zeros((NP, F), jnp.float32)

    aggp, degp = _sc_agg(xp, srcp, dstp, zeros_np)
    s2, h = _mid(aggp, degp.reshape(NC, NP, 1), xp, W1l, b1, W1r, W2l)
    parts = _sc_scalar(s2.reshape(NP), srcp, dstp)
    t2 = _mid2(h, W2r, b2)
    res = _final(parts.reshape(NW, GR, 128),
                 degp.reshape(NC, GR, 128), t2.reshape(GR, 128))
    return res.reshape(NP, 1)[:N_NODES]
